# Initial kernel scaffold; baseline (speedup 1.0000x reference)
import jax
import jax.numpy as jnp
from jax.experimental import pallas as pl


def kernel(embs, fc_w, fc_b, s1_w, s1_b, s2_w, s2_b, pos_w, neg_w, anchors, pos_idx, neg_idx, sta_pos_idx, sta_neg_idx):
    raise NotImplementedError("write your pallas kernel here")


# speedup vs baseline: 1.7853x; 1.7853x over previous
import functools

import jax
import jax.numpy as jnp
from jax import lax
from jax.experimental import pallas as pl
from jax.experimental.pallas import tpu as pltpu
from jax.experimental.pallas import tpu_sc as plsc

_N, _D, _B, _K = 100000, 32, 1024, 32
_M = _B * _K
_NW = 32
_PW = _M // _NW
_CH = 128
_NCH = _PW // _CH
_AW = _B // _NW


def _sc_gather(embs, idx_p, idx_n, idx_sp, idx_sn, idx_a):
    mesh = plsc.VectorSubcoreMesh(core_axis_name="c", subcore_axis_name="s")
    row_t = jax.ShapeDtypeStruct((_M, _D), jnp.float32)
    z_t = jax.ShapeDtypeStruct((_B, _D), jnp.float32)

    @functools.partial(
        pl.kernel,
        out_type=(row_t, row_t, row_t, row_t, z_t),
        mesh=mesh,
        scratch_types=[
            pltpu.VMEM((_PW,), jnp.int32),
            pltpu.VMEM((_AW,), jnp.int32),
            pltpu.VMEM((_CH, _D), jnp.float32),
            pltpu.VMEM((_AW, _D), jnp.float32),
            pltpu.SemaphoreType.DMA,
        ],
        compiler_params=pltpu.CompilerParams(use_tc_tiling_on_sc=False),
    )
    def k(table, ip, ng, isp, isn, ia, op, on, osp, osn, oz,
          idx_v, av, buf, zbuf, gsem):
        wid = lax.axis_index("s") * 2 + lax.axis_index("c")
        abase = wid * _AW
        pltpu.sync_copy(ia.at[pl.ds(abase, _AW)], av)
        pltpu.async_copy(table.at[av], zbuf, gsem).wait()
        pltpu.sync_copy(zbuf, oz.at[pl.ds(abase, _AW)])
        base = wid * _PW
        for iref, oref in ((ip, op), (ng, on), (isp, osp), (isn, osn)):
            pltpu.sync_copy(iref.at[pl.ds(base, _PW)], idx_v)

            def step(j, _, iref=iref, oref=oref):
                pltpu.async_copy(
                    table.at[idx_v.at[pl.ds(j * _CH, _CH)]], buf, gsem
                ).wait()
                pltpu.sync_copy(buf, oref.at[pl.ds(base + j * _CH, _CH)])
                return 0

            lax.fori_loop(0, _NCH, step, 0)

    return k(embs, idx_p, idx_n, idx_sp, idx_sn, idx_a)


def _tc_loss(z1, p3, n3, sp3, sn3, pw, nw, wmat, bvec):
    bblk = 128
    grid = (_B // bblk,)

    def body(wm_ref, bv_ref, z_ref, p_ref, n_ref, sp_ref, sn_ref,
             pw_ref, nw_ref, out_ref):
        fc1 = wm_ref[0:1, :]
        fc2 = wm_ref[1:2, :]
        s1w = wm_ref[2:3, :]
        s2w = wm_ref[3:4, :]
        fcb, s1b, s2b = bv_ref[0], bv_ref[1], bv_ref[2]
        z1b = z_ref[...]
        rs_z = jnp.sum(z1b * fc1, axis=1) + fcb
        s1 = jax.nn.sigmoid(jnp.sum(z1b * s1w, axis=1) + s1b)

        def side(emb3, sta3, w2d, positive):
            sgn = -1.0 if positive else 1.0
            logit = jnp.sum(emb3 * z1b[:, None, :], axis=-1)
            l = jnp.mean(jax.nn.softplus(sgn * logit), axis=1)
            rs = jnp.sum(sta3 * fc2[None, :, :], axis=-1) + rs_z[:, None]
            l = l + jnp.mean(w2d * jax.nn.softplus(sgn * rs), axis=1)
            s2 = jax.nn.sigmoid(jnp.sum(sta3 * s2w[None, :, :], axis=-1) + s2b)
            diff = s1[:, None] - s2
            q = jnp.minimum(diff, -0.5) if positive else jnp.maximum(diff, 0.5)
            tmp = q - diff
            return l + 5.0 * jnp.sum(tmp * tmp, axis=1)

        lp = side(p_ref[...], sp_ref[...], pw_ref[...], True)
        ln = side(n_ref[...], sn_ref[...], nw_ref[...], False)
        part = jnp.sum(lp + ln)

        @pl.when(pl.program_id(0) == 0)
        def _():
            out_ref[0, 0] = 0.0

        out_ref[0, 0] += part

    tbl_spec = pl.BlockSpec((bblk, _K, _D), lambda i: (i, 0, 0))
    return pl.pallas_call(
        body,
        grid=grid,
        in_specs=[
            pl.BlockSpec((4, _D), lambda i: (0, 0)),
            pl.BlockSpec(memory_space=pltpu.SMEM),
            pl.BlockSpec((bblk, _D), lambda i: (i, 0)),
            tbl_spec, tbl_spec, tbl_spec, tbl_spec,
            pl.BlockSpec((bblk, _K), lambda i: (i, 0)),
            pl.BlockSpec((bblk, _K), lambda i: (i, 0)),
        ],
        out_specs=pl.BlockSpec(memory_space=pltpu.SMEM),
        out_shape=jax.ShapeDtypeStruct((1, 1), jnp.float32),
    )(wmat, bvec, z1, p3, n3, sp3, sn3, pw, nw)


def kernel(embs, fc_w, fc_b, s1_w, s1_b, s2_w, s2_b, pos_w, neg_w,
           anchors, pos_idx, neg_idx, sta_pos_idx, sta_neg_idx):
    i32 = jnp.int32
    rp, rn, rsp, rsn, z1 = _sc_gather(
        embs,
        pos_idx.reshape(-1).astype(i32),
        neg_idx.reshape(-1).astype(i32),
        sta_pos_idx.reshape(-1).astype(i32),
        sta_neg_idx.reshape(-1).astype(i32),
        anchors.astype(i32),
    )
    p3 = rp.reshape(_B, _K, _D)
    n3 = rn.reshape(_B, _K, _D)
    sp3 = rsp.reshape(_B, _K, _D)
    sn3 = rsn.reshape(_B, _K, _D)
    wmat = jnp.concatenate([fc_w[:, :_D], fc_w[:, _D:], s1_w, s2_w], axis=0)
    bvec = jnp.stack([fc_b[0], s1_b[0], s2_b[0]])
    out = _tc_loss(z1, p3, n3, sp3, sn3, pos_w, neg_w, wmat, bvec)
    return out[0, 0]


# speedup vs baseline: 2.1723x; 1.2168x over previous
import functools

import jax
import jax.numpy as jnp
from jax import lax
from jax.experimental import pallas as pl
from jax.experimental.pallas import tpu as pltpu
from jax.experimental.pallas import tpu_sc as plsc

_N, _D, _B, _K = 100000, 32, 1024, 32
_M = _B * _K
_NW = 32
_AW = _B // _NW
_PW = _AW * _K


def _sc_fused(embs, idx_p, idx_n, idx_sp, idx_sn, idx_a, warr):
    mesh = plsc.VectorSubcoreMesh(core_axis_name="c", subcore_axis_name="s")
    vec_t = jax.ShapeDtypeStruct((_M,), jnp.float32)

    @functools.partial(
        pl.kernel,
        out_type=tuple(vec_t for _ in range(7)),
        mesh=mesh,
        scratch_types=[
            pltpu.VMEM((68 * 16,), jnp.float32),
            pltpu.VMEM((_AW,), jnp.int32),
            pltpu.VMEM((_AW, _D), jnp.float32),
            pltpu.VMEM((_PW,), jnp.int32),
            pltpu.VMEM((_PW,), jnp.int32),
            pltpu.VMEM((_PW,), jnp.int32),
            pltpu.VMEM((_PW,), jnp.int32),
            pltpu.VMEM((_K, _D), jnp.float32),
            pltpu.VMEM((_K, _D), jnp.float32),
            pltpu.VMEM((_K, _D), jnp.float32),
            pltpu.VMEM((_K, _D), jnp.float32),
            pltpu.VMEM((7, _PW), jnp.float32),
            pltpu.SemaphoreType.DMA,
        ],
        compiler_params=pltpu.CompilerParams(
            use_tc_tiling_on_sc=False, needs_layout_passes=False),
    )
    def k(table, ip, ng, isp, isn, ia, wh,
          o_lp, o_ln, o_rsp, o_rsn, o_s2p, o_s2n, o_s1,
          wv, av, zbuf, ixp, ixn, ixsp, ixsn,
          bufp, bufn, bufsp, bufsn, obuf, gsem):
        wid = lax.axis_index("s") * 2 + lax.axis_index("c")
        pltpu.sync_copy(wh, wv)
        abase = wid * _AW
        pltpu.sync_copy(ia.at[pl.ds(abase, _AW)], av)
        pltpu.async_copy(table.at[av], zbuf, gsem).wait()
        base = wid * _PW
        pltpu.sync_copy(ip.at[pl.ds(base, _PW)], ixp)
        pltpu.sync_copy(ng.at[pl.ds(base, _PW)], ixn)
        pltpu.sync_copy(isp.at[pl.ds(base, _PW)], ixsp)
        pltpu.sync_copy(isn.at[pl.ds(base, _PW)], ixsn)

        fc1_0 = wv[pl.ds(64 * 16, 16)]
        fc1_1 = wv[pl.ds(65 * 16, 16)]
        s1w_0 = wv[pl.ds(66 * 16, 16)]
        s1w_1 = wv[pl.ds(67 * 16, 16)]
        r16 = lax.iota(jnp.int32, 16)
        rows = (r16, r16 + 16)
        zero = jnp.zeros((16,), jnp.float32)

        def anchor_step(a, _):
            o = a * _K
            cps = [
                pltpu.async_copy(table.at[ix.at[pl.ds(o, _K)]], buf, gsem)
                for ix, buf in ((ixp, bufp), (ixn, bufn),
                                (ixsp, bufsp), (ixsn, bufsn))
            ]
            af = jnp.full((16,), a, jnp.int32)
            z0 = plsc.load_gather(zbuf, [af, rows[0]])
            z1v = plsc.load_gather(zbuf, [af, rows[1]])
            rs_z = jnp.sum(z0 * fc1_0 + z1v * fc1_1)
            s1_d = jnp.sum(z0 * s1w_0 + z1v * s1w_1)
            for cp in cps:
                cp.wait()
            acc = [[zero] * 2 for _ in range(6)]
            for d in range(_D):
                df = jnp.full((16,), d, jnp.int32)
                zc = plsc.load_gather(zbuf, [af, df])
                fc2d = wv[pl.ds(d * 16, 16)]
                s2wd = wv[pl.ds((32 + d) * 16, 16)]
                for h in range(2):
                    rh = rows[h]
                    cp_ = plsc.load_gather(bufp, [rh, df])
                    acc[0][h] = acc[0][h] + cp_ * zc
                    cn_ = plsc.load_gather(bufn, [rh, df])
                    acc[1][h] = acc[1][h] + cn_ * zc
                    sp_ = plsc.load_gather(bufsp, [rh, df])
                    acc[2][h] = acc[2][h] + sp_ * fc2d
                    acc[4][h] = acc[4][h] + sp_ * s2wd
                    sn_ = plsc.load_gather(bufsn, [rh, df])
                    acc[3][h] = acc[3][h] + sn_ * fc2d
                    acc[5][h] = acc[5][h] + sn_ * s2wd
            s1_v = jnp.full((16,), s1_d)
            for h in range(2):
                off = o + 16 * h
                obuf[0, pl.ds(off, 16)] = acc[0][h]
                obuf[1, pl.ds(off, 16)] = acc[1][h]
                obuf[2, pl.ds(off, 16)] = acc[2][h] + rs_z
                obuf[3, pl.ds(off, 16)] = acc[3][h] + rs_z
                obuf[4, pl.ds(off, 16)] = acc[4][h]
                obuf[5, pl.ds(off, 16)] = acc[5][h]
                obuf[6, pl.ds(off, 16)] = s1_v
            return 0

        lax.fori_loop(0, _AW, anchor_step, 0)
        outs = (o_lp, o_ln, o_rsp, o_rsn, o_s2p, o_s2n, o_s1)
        for i, oref in enumerate(outs):
            pltpu.sync_copy(obuf.at[i], oref.at[pl.ds(base, _PW)])

    return k(embs, idx_p, idx_n, idx_sp, idx_sn, idx_a, warr)


def _tc_loss(lp, ln, rsp, rsn, s2p, s2n, s1d, pw, nw, bvec):
    def body(bv_ref, lp_ref, ln_ref, rsp_ref, rsn_ref, s2p_ref, s2n_ref,
             s1_ref, pw_ref, nw_ref, out_ref):
        fcb, s1b, s2b = bv_ref[0], bv_ref[1], bv_ref[2]
        sp = jax.nn.softplus
        sg = jax.nn.sigmoid
        invk = 1.0 / _K
        s1 = sg(s1_ref[...] + s1b)
        dp = s1 - sg(s2p_ref[...] + s2b)
        tp = jnp.minimum(dp, -0.5) - dp
        dn = s1 - sg(s2n_ref[...] + s2b)
        tn = jnp.maximum(dn, 0.5) - dn
        term = (sp(-lp_ref[...]) + pw_ref[...] * sp(-(rsp_ref[...] + fcb))
                + sp(ln_ref[...]) + nw_ref[...] * sp(rsn_ref[...] + fcb))
        total = invk * jnp.sum(term) + 5.0 * jnp.sum(tp * tp + tn * tn)
        out_ref[0, 0] = total

    vspec = pl.BlockSpec((_M // 128, 128), lambda: (0, 0))
    return pl.pallas_call(
        body,
        in_specs=[pl.BlockSpec(memory_space=pltpu.SMEM)] + [vspec] * 9,
        out_specs=pl.BlockSpec(memory_space=pltpu.SMEM),
        out_shape=jax.ShapeDtypeStruct((1, 1), jnp.float32),
    )(bvec, lp, ln, rsp, rsn, s2p, s2n, s1d, pw, nw)


def kernel(embs, fc_w, fc_b, s1_w, s1_b, s2_w, s2_b, pos_w, neg_w,
           anchors, pos_idx, neg_idx, sta_pos_idx, sta_neg_idx):
    i32 = jnp.int32
    fc1 = fc_w[0, :_D]
    fc2 = fc_w[0, _D:]
    wsplat = jnp.broadcast_to(
        jnp.concatenate([fc2, s2_w[0]])[:, None], (64, 16)).reshape(-1)
    warr = jnp.concatenate([wsplat, fc1, s1_w[0]])
    outs = _sc_fused(
        embs,
        pos_idx.reshape(-1).astype(i32),
        neg_idx.reshape(-1).astype(i32),
        sta_pos_idx.reshape(-1).astype(i32),
        sta_neg_idx.reshape(-1).astype(i32),
        anchors.astype(i32),
        warr,
    )
    shaped = [o.reshape(_M // 128, 128) for o in outs]
    bvec = jnp.stack([fc_b[0], s1_b[0], s2_b[0]])
    out = _tc_loss(*shaped,
                   pos_w.reshape(_M // 128, 128),
                   neg_w.reshape(_M // 128, 128), bvec)
    return out[0, 0]


# speedup vs baseline: 2.3600x; 1.0864x over previous
import functools

import jax
import jax.numpy as jnp
from jax import lax
from jax.experimental import pallas as pl
from jax.experimental.pallas import tpu as pltpu
from jax.experimental.pallas import tpu_sc as plsc

_N, _D, _B, _K = 100000, 32, 1024, 32
_M = _B * _K
_NW = 32
_AW = _B // _NW
_PW = _AW * _K


def _sc_fused(embs, idx_p, idx_n, idx_sp, idx_sn, idx_a, warr):
    mesh = plsc.VectorSubcoreMesh(core_axis_name="c", subcore_axis_name="s")
    vec_t = jax.ShapeDtypeStruct((_M,), jnp.float32)
    grp = 2
    grows = grp * _K
    ngrp = _AW // grp

    @functools.partial(
        pl.kernel,
        out_type=tuple(vec_t for _ in range(7)),
        mesh=mesh,
        scratch_types=[
            pltpu.VMEM((68 * 16,), jnp.float32),
            pltpu.VMEM((_AW,), jnp.int32),
            pltpu.VMEM((_AW, _D), jnp.float32),
            pltpu.VMEM((_PW,), jnp.int32),
            pltpu.VMEM((_PW,), jnp.int32),
            pltpu.VMEM((_PW,), jnp.int32),
            pltpu.VMEM((_PW,), jnp.int32),
            pltpu.VMEM((2, 4, grows, _D), jnp.float32),
            pltpu.VMEM((7, _PW), jnp.float32),
            pltpu.SemaphoreType.DMA,
            pltpu.SemaphoreType.DMA,
        ],
        compiler_params=pltpu.CompilerParams(
            use_tc_tiling_on_sc=False, needs_layout_passes=False),
    )
    def k(table, ip, ng, isp, isn, ia, wh,
          o_lp, o_ln, o_rsp, o_rsn, o_s2p, o_s2n, o_s1,
          wv, av, zbuf, ixp, ixn, ixsp, ixsn, bufs, obuf, sem0, sem1):
        wid = lax.axis_index("s") * 2 + lax.axis_index("c")
        pltpu.sync_copy(wh, wv)
        abase = wid * _AW
        pltpu.sync_copy(ia.at[pl.ds(abase, _AW)], av)
        pltpu.async_copy(table.at[av], zbuf, sem0).wait()
        base = wid * _PW
        pltpu.sync_copy(ip.at[pl.ds(base, _PW)], ixp)
        pltpu.sync_copy(ng.at[pl.ds(base, _PW)], ixn)
        pltpu.sync_copy(isp.at[pl.ds(base, _PW)], ixsp)
        pltpu.sync_copy(isn.at[pl.ds(base, _PW)], ixsn)
        ixs = (ixp, ixn, ixsp, ixsn)

        fc1_0 = wv[pl.ds(64 * 16, 16)]
        fc1_1 = wv[pl.ds(65 * 16, 16)]
        s1w_0 = wv[pl.ds(66 * 16, 16)]
        s1w_1 = wv[pl.ds(67 * 16, 16)]
        r16 = lax.iota(jnp.int32, 16)
        halves = (r16, r16 + 16)
        zero = jnp.zeros((16,), jnp.float32)
        sems = (sem0, sem1)

        def fire(g, slot):
            o = g * grows
            for t in range(4):
                pltpu.async_copy(
                    table.at[ixs[t].at[pl.ds(o, grows)]],
                    bufs.at[slot, t], sems[slot])

        def drain(g, slot):
            o = g * grows
            for t in range(4):
                pltpu.make_async_copy(
                    table.at[ixs[t].at[pl.ds(o, grows)]],
                    bufs.at[slot, t], sems[slot]).wait()

        def compute(g, slot):
            for a_local in range(grp):
                a = g * grp + a_local
                af = jnp.full((16,), a, jnp.int32)
                z0 = plsc.load_gather(zbuf, [af, halves[0]])
                z1v = plsc.load_gather(zbuf, [af, halves[1]])
                rs_z = jnp.sum(z0 * fc1_0 + z1v * fc1_1)
                s1_d = jnp.sum(z0 * s1w_0 + z1v * s1w_1)
                rows = (halves[0] + a_local * _K, halves[1] + a_local * _K)
                acc = [[zero] * 2 for _ in range(6)]
                for d in range(_D):
                    df = jnp.full((16,), d, jnp.int32)
                    zc = plsc.load_gather(zbuf, [af, df])
                    fc2d = wv[pl.ds(d * 16, 16)]
                    s2wd = wv[pl.ds((32 + d) * 16, 16)]
                    for h in range(2):
                        rh = rows[h]
                        cp_ = plsc.load_gather(bufs.at[slot, 0], [rh, df])
                        acc[0][h] = acc[0][h] + cp_ * zc
                        cn_ = plsc.load_gather(bufs.at[slot, 1], [rh, df])
                        acc[1][h] = acc[1][h] + cn_ * zc
                        sp_ = plsc.load_gather(bufs.at[slot, 2], [rh, df])
                        acc[2][h] = acc[2][h] + sp_ * fc2d
                        acc[4][h] = acc[4][h] + sp_ * s2wd
                        sn_ = plsc.load_gather(bufs.at[slot, 3], [rh, df])
                        acc[3][h] = acc[3][h] + sn_ * fc2d
                        acc[5][h] = acc[5][h] + sn_ * s2wd
                s1_v = jnp.full((16,), s1_d)
                for h in range(2):
                    off = a * _K + 16 * h
                    obuf[0, pl.ds(off, 16)] = acc[0][h]
                    obuf[1, pl.ds(off, 16)] = acc[1][h]
                    obuf[2, pl.ds(off, 16)] = acc[2][h] + rs_z
                    obuf[3, pl.ds(off, 16)] = acc[3][h] + rs_z
                    obuf[4, pl.ds(off, 16)] = acc[4][h]
                    obuf[5, pl.ds(off, 16)] = acc[5][h]
                    obuf[6, pl.ds(off, 16)] = s1_v

        fire(0, 0)

        def pair_step(p, _):
            g0 = 2 * p
            fire(g0 + 1, 1)
            drain(g0, 0)
            compute(g0, 0)

            @pl.when(g0 + 2 < ngrp)
            def _():
                fire(g0 + 2, 0)

            drain(g0 + 1, 1)
            compute(g0 + 1, 1)
            return 0

        lax.fori_loop(0, ngrp // 2, pair_step, 0)
        outs = (o_lp, o_ln, o_rsp, o_rsn, o_s2p, o_s2n, o_s1)
        for i, oref in enumerate(outs):
            pltpu.sync_copy(obuf.at[i], oref.at[pl.ds(base, _PW)])

    return k(embs, idx_p, idx_n, idx_sp, idx_sn, idx_a, warr)


def _tc_loss(lp, ln, rsp, rsn, s2p, s2n, s1d, pw, nw, bvec):
    def body(bv_ref, lp_ref, ln_ref, rsp_ref, rsn_ref, s2p_ref, s2n_ref,
             s1_ref, pw_ref, nw_ref, out_ref):
        fcb, s1b, s2b = bv_ref[0], bv_ref[1], bv_ref[2]
        sp = jax.nn.softplus
        sg = jax.nn.sigmoid
        invk = 1.0 / _K
        s1 = sg(s1_ref[...] + s1b)
        dp = s1 - sg(s2p_ref[...] + s2b)
        tp = jnp.minimum(dp, -0.5) - dp
        dn = s1 - sg(s2n_ref[...] + s2b)
        tn = jnp.maximum(dn, 0.5) - dn
        term = (sp(-lp_ref[...]) + pw_ref[...] * sp(-(rsp_ref[...] + fcb))
                + sp(ln_ref[...]) + nw_ref[...] * sp(rsn_ref[...] + fcb))
        total = invk * jnp.sum(term) + 5.0 * jnp.sum(tp * tp + tn * tn)
        out_ref[0, 0] = total

    vspec = pl.BlockSpec((_M // 128, 128), lambda: (0, 0))
    return pl.pallas_call(
        body,
        in_specs=[pl.BlockSpec(memory_space=pltpu.SMEM)] + [vspec] * 9,
        out_specs=pl.BlockSpec(memory_space=pltpu.SMEM),
        out_shape=jax.ShapeDtypeStruct((1, 1), jnp.float32),
    )(bvec, lp, ln, rsp, rsn, s2p, s2n, s1d, pw, nw)


def kernel(embs, fc_w, fc_b, s1_w, s1_b, s2_w, s2_b, pos_w, neg_w,
           anchors, pos_idx, neg_idx, sta_pos_idx, sta_neg_idx):
    i32 = jnp.int32
    fc1 = fc_w[0, :_D]
    fc2 = fc_w[0, _D:]
    wsplat = jnp.broadcast_to(
        jnp.concatenate([fc2, s2_w[0]])[:, None], (64, 16)).reshape(-1)
    warr = jnp.concatenate([wsplat, fc1, s1_w[0]])
    outs = _sc_fused(
        embs,
        pos_idx.reshape(-1).astype(i32),
        neg_idx.reshape(-1).astype(i32),
        sta_pos_idx.reshape(-1).astype(i32),
        sta_neg_idx.reshape(-1).astype(i32),
        anchors.astype(i32),
        warr,
    )
    shaped = [o.reshape(_M // 128, 128) for o in outs]
    bvec = jnp.stack([fc_b[0], s1_b[0], s2_b[0]])
    out = _tc_loss(*shaped,
                   pos_w.reshape(_M // 128, 128),
                   neg_w.reshape(_M // 128, 128), bvec)
    return out[0, 0]


# speedup vs baseline: 3.2069x; 1.3588x over previous
import functools

import jax
import jax.numpy as jnp
from jax import lax
from jax.experimental import pallas as pl
from jax.experimental.pallas import tpu as pltpu
from jax.experimental.pallas import tpu_sc as plsc

_N, _D, _B, _K = 100000, 32, 1024, 32
_M = _B * _K
_NW = 32
_AW = _B // _NW
_PW = _AW * _K


_PITCH = 65


def _sc_fused(embs, idx_p, idx_n, idx_sp, idx_sn, idx_a, warr):
    mesh = plsc.VectorSubcoreMesh(core_axis_name="c", subcore_axis_name="s")
    vec_t = jax.ShapeDtypeStruct((_M,), jnp.float32)
    grp = 2
    grows = grp * _K
    ngrp = _AW // grp

    tsz = _D * _PITCH + grows

    @functools.partial(
        pl.kernel,
        out_type=tuple(vec_t for _ in range(7)),
        mesh=mesh,
        scratch_types=[
            pltpu.VMEM((128,), jnp.float32),
            pltpu.VMEM((_AW,), jnp.int32),
            pltpu.VMEM((_AW, _D), jnp.float32),
            pltpu.VMEM((_PW,), jnp.int32),
            pltpu.VMEM((_PW,), jnp.int32),
            pltpu.VMEM((_PW,), jnp.int32),
            pltpu.VMEM((_PW,), jnp.int32),
            pltpu.VMEM((2, 4, grows, _D), jnp.float32),
            pltpu.VMEM((4, tsz), jnp.float32),
            pltpu.VMEM((7, _PW), jnp.float32),
            pltpu.SemaphoreType.DMA,
            pltpu.SemaphoreType.DMA,
        ],
        compiler_params=pltpu.CompilerParams(
            use_tc_tiling_on_sc=False, needs_layout_passes=False),
    )
    def k(table, ip, ng, isp, isn, ia, wh,
          o_lp, o_ln, o_rsp, o_rsn, o_s2p, o_s2n, o_s1,
          wv, av, zbuf, ixp, ixn, ixsp, ixsn, bufs, tbuf, obuf, sem0, sem1):
        wid = lax.axis_index("s") * 2 + lax.axis_index("c")
        pltpu.sync_copy(wh, wv)
        abase = wid * _AW
        pltpu.sync_copy(ia.at[pl.ds(abase, _AW)], av)
        pltpu.async_copy(table.at[av], zbuf, sem0).wait()
        base = wid * _PW
        pltpu.sync_copy(ip.at[pl.ds(base, _PW)], ixp)
        pltpu.sync_copy(ng.at[pl.ds(base, _PW)], ixn)
        pltpu.sync_copy(isp.at[pl.ds(base, _PW)], ixsp)
        pltpu.sync_copy(isn.at[pl.ds(base, _PW)], ixsn)
        ixs = (ixp, ixn, ixsp, ixsn)

        fc1 = (wv[pl.ds(0, 16)], wv[pl.ds(16, 16)])
        fc2 = (wv[pl.ds(32, 16)], wv[pl.ds(48, 16)])
        s1w = (wv[pl.ds(64, 16)], wv[pl.ds(80, 16)])
        s2w = (wv[pl.ds(96, 16)], wv[pl.ds(112, 16)])
        r16 = lax.iota(jnp.int32, 16)
        halves = (r16, r16 + 16)
        i_pitch = r16 * _PITCH
        zero = jnp.zeros((16,), jnp.float32)
        sems = (sem0, sem1)

        def fire(g, slot):
            o = g * grows
            for t in range(4):
                pltpu.async_copy(
                    table.at[ixs[t].at[pl.ds(o, grows)]],
                    bufs.at[slot, t], sems[slot])

        def drain(g, slot):
            o = g * grows
            for t in range(4):
                pltpu.make_async_copy(
                    table.at[ixs[t].at[pl.ds(o, grows)]],
                    bufs.at[slot, t], sems[slot]).wait()

        def transpose(slot):
            def t_step(r, _):
                rf = jnp.full((16,), r, jnp.int32)
                for h in range(2):
                    dst = i_pitch + (jnp.full((16,), h * 16 * _PITCH,
                                              jnp.int32) + rf)
                    for t in range(4):
                        v = plsc.load_gather(bufs.at[slot, t],
                                             [rf, halves[h]])
                        plsc.store_scatter(tbuf.at[t], [dst], v)
                return 0

            lax.fori_loop(0, grows, t_step, 0)

        gdn = lax.GatherDimensionNumbers(
            offset_dims=(), collapsed_slice_dims=(0,), start_index_map=(0,))

        def take16(vpair, d):
            src = vpair[d // 16]
            idx = jnp.full((16, 1), d % 16, jnp.int32)
            return lax.gather(src, idx, gdn, slice_sizes=(1,),
                              mode=lax.GatherScatterMode.PROMISE_IN_BOUNDS)

        def compute(g, slot):
            transpose(slot)
            for a_local in range(grp):
                a = g * grp + a_local
                af = jnp.full((16,), a, jnp.int32)
                z0 = plsc.load_gather(zbuf, [af, halves[0]])
                z1v = plsc.load_gather(zbuf, [af, halves[1]])
                rs_z = jnp.sum(z0 * fc1[0] + z1v * fc1[1])
                s1_d = jnp.sum(z0 * s1w[0] + z1v * s1w[1])
                zpair = (z0, z1v)
                acc = [[zero] * 2 for _ in range(6)]
                for d in range(_D):
                    zc = take16(zpair, d)
                    fc2d = take16(fc2, d)
                    s2wd = take16(s2w, d)
                    for h in range(2):
                        off = d * _PITCH + a_local * _K + h * 16
                        cp_ = tbuf[0, pl.ds(off, 16)]
                        acc[0][h] = acc[0][h] + cp_ * zc
                        cn_ = tbuf[1, pl.ds(off, 16)]
                        acc[1][h] = acc[1][h] + cn_ * zc
                        sp_ = tbuf[2, pl.ds(off, 16)]
                        acc[2][h] = acc[2][h] + sp_ * fc2d
                        acc[4][h] = acc[4][h] + sp_ * s2wd
                        sn_ = tbuf[3, pl.ds(off, 16)]
                        acc[3][h] = acc[3][h] + sn_ * fc2d
                        acc[5][h] = acc[5][h] + sn_ * s2wd
                s1_v = jnp.full((16,), s1_d)
                for h in range(2):
                    off = a * _K + 16 * h
                    obuf[0, pl.ds(off, 16)] = acc[0][h]
                    obuf[1, pl.ds(off, 16)] = acc[1][h]
                    obuf[2, pl.ds(off, 16)] = acc[2][h] + rs_z
                    obuf[3, pl.ds(off, 16)] = acc[3][h] + rs_z
                    obuf[4, pl.ds(off, 16)] = acc[4][h]
                    obuf[5, pl.ds(off, 16)] = acc[5][h]
                    obuf[6, pl.ds(off, 16)] = s1_v

        fire(0, 0)

        def pair_step(p, _):
            g0 = 2 * p
            fire(g0 + 1, 1)
            drain(g0, 0)
            compute(g0, 0)

            @pl.when(g0 + 2 < ngrp)
            def _():
                fire(g0 + 2, 0)

            drain(g0 + 1, 1)
            compute(g0 + 1, 1)
            return 0

        lax.fori_loop(0, ngrp // 2, pair_step, 0)
        outs = (o_lp, o_ln, o_rsp, o_rsn, o_s2p, o_s2n, o_s1)
        for i, oref in enumerate(outs):
            pltpu.sync_copy(obuf.at[i], oref.at[pl.ds(base, _PW)])

    return k(embs, idx_p, idx_n, idx_sp, idx_sn, idx_a, warr)


def _tc_loss(lp, ln, rsp, rsn, s2p, s2n, s1d, pw, nw, bvec):
    def body(bv_ref, lp_ref, ln_ref, rsp_ref, rsn_ref, s2p_ref, s2n_ref,
             s1_ref, pw_ref, nw_ref, out_ref):
        fcb, s1b, s2b = bv_ref[0], bv_ref[1], bv_ref[2]
        sp = jax.nn.softplus
        sg = jax.nn.sigmoid
        invk = 1.0 / _K
        s1 = sg(s1_ref[...] + s1b)
        dp = s1 - sg(s2p_ref[...] + s2b)
        tp = jnp.minimum(dp, -0.5) - dp
        dn = s1 - sg(s2n_ref[...] + s2b)
        tn = jnp.maximum(dn, 0.5) - dn
        term = (sp(-lp_ref[...]) + pw_ref[...] * sp(-(rsp_ref[...] + fcb))
                + sp(ln_ref[...]) + nw_ref[...] * sp(rsn_ref[...] + fcb))
        total = invk * jnp.sum(term) + 5.0 * jnp.sum(tp * tp + tn * tn)
        out_ref[0, 0] = total

    vspec = pl.BlockSpec((_M // 128, 128), lambda: (0, 0))
    return pl.pallas_call(
        body,
        in_specs=[pl.BlockSpec(memory_space=pltpu.SMEM)] + [vspec] * 9,
        out_specs=pl.BlockSpec(memory_space=pltpu.SMEM),
        out_shape=jax.ShapeDtypeStruct((1, 1), jnp.float32),
    )(bvec, lp, ln, rsp, rsn, s2p, s2n, s1d, pw, nw)


def kernel(embs, fc_w, fc_b, s1_w, s1_b, s2_w, s2_b, pos_w, neg_w,
           anchors, pos_idx, neg_idx, sta_pos_idx, sta_neg_idx):
    i32 = jnp.int32
    warr = jnp.concatenate([fc_w[0, :_D], fc_w[0, _D:], s1_w[0], s2_w[0]])
    outs = _sc_fused(
        embs,
        pos_idx.reshape(-1).astype(i32),
        neg_idx.reshape(-1).astype(i32),
        sta_pos_idx.reshape(-1).astype(i32),
        sta_neg_idx.reshape(-1).astype(i32),
        anchors.astype(i32),
        warr,
    )
    shaped = [o.reshape(_M // 128, 128) for o in outs]
    bvec = jnp.stack([fc_b[0], s1_b[0], s2_b[0]])
    out = _tc_loss(*shaped,
                   pos_w.reshape(_M // 128, 128),
                   neg_w.reshape(_M // 128, 128), bvec)
    return out[0, 0]
